# Initial kernel scaffold; baseline (speedup 1.0000x reference)
import jax
import jax.numpy as jnp
from jax.experimental import pallas as pl


def kernel(x, edge_index, batch_index, W1, b1, W2, b2, W3, b3):
    raise NotImplementedError("write your pallas kernel here")


# speedup vs baseline: 40.2372x; 40.2372x over previous
import functools

import jax
import jax.numpy as jnp
from jax import lax
from jax.experimental import pallas as pl
from jax.experimental.pallas import tpu as pltpu
from jax.experimental.pallas import tpu_sc as plsc

_NC = 2
_NS = 16
_LANES = 128
_K = 8
_F = 16


def _cdiv(a, b):
    return -(-a // b)


@functools.lru_cache(maxsize=None)
def _seg_make(n_pad, rpt_e, rpt_n, feat, interpret=False):
    f32 = jnp.float32
    mesh = plsc.VectorSubcoreMesh(core_axis_name="c", subcore_axis_name="s",
                                  num_cores=_NC, num_subcores=_NS)
    cpt = rpt_e // _K

    def body(hn, src2, dst2, zz, p0, p1, acc, src_v, dst_v, rows_v, sem):
        cid = lax.axis_index("c")
        sid = lax.axis_index("s")
        nsl = pl.ds(sid * rpt_n, rpt_n)
        pltpu.sync_copy(zz.at[nsl], acc.at[nsl])
        plsc.subcore_barrier()
        row0 = (cid * _NS + sid) * rpt_e

        def chunk(g, carry):
            base = row0 + g * _K
            pltpu.sync_copy(src2.at[pl.ds(base, _K)], src_v)
            pltpu.sync_copy(dst2.at[pl.ds(base, _K)], dst_v)
            descs = [pltpu.async_copy(hn.at[src_v.at[j]], rows_v.at[j], sem)
                     for j in range(_K)]
            for j in range(_K):
                descs[j].wait()
                pltpu.sync_copy(rows_v.at[j], acc.at[dst_v.at[j]], add=True)
            return carry

        lax.fori_loop(0, cpt, chunk, 0)
        plsc.subcore_barrier()

        @pl.when(cid == 0)
        def _():
            pltpu.sync_copy(acc.at[nsl], p0.at[nsl])

        @pl.when(cid == 1)
        def _():
            pltpu.sync_copy(acc.at[nsl], p1.at[nsl])

    return pl.kernel(
        body,
        out_type=(jax.ShapeDtypeStruct((n_pad, feat), f32),
                  jax.ShapeDtypeStruct((n_pad, feat), f32)),
        mesh=mesh,
        scratch_types=(pltpu.VMEM_SHARED((n_pad, feat), f32),
                       pltpu.VMEM((_K, _LANES), jnp.int32),
                       pltpu.VMEM((_K, _LANES), jnp.int32),
                       pltpu.VMEM((_K, _LANES, feat), f32),
                       pltpu.SemaphoreType.DMA),
        compiler_params=pltpu.CompilerParams(use_tc_tiling_on_sc=False),
        interpret=interpret)


@functools.lru_cache(maxsize=None)
def _deg_make(n_pad, rpt_e, rpt_n, interpret=False):
    f32 = jnp.float32
    mesh = plsc.VectorSubcoreMesh(core_axis_name="c", subcore_axis_name="s",
                                  num_cores=_NC, num_subcores=_NS)
    cpt = rpt_e // _K

    def body(dst2, zd, ones, q0, q1, accd, dst_v, ones_v, sem):
        cid = lax.axis_index("c")
        sid = lax.axis_index("s")
        nsl = pl.ds(sid * rpt_n, rpt_n)
        pltpu.sync_copy(zd.at[nsl], accd.at[nsl])
        pltpu.sync_copy(ones, ones_v)
        plsc.subcore_barrier()
        row0 = (cid * _NS + sid) * rpt_e

        def chunk(g, carry):
            base = row0 + g * _K
            pltpu.sync_copy(dst2.at[pl.ds(base, _K)], dst_v)
            descs = [pltpu.async_copy(ones_v, accd.at[dst_v.at[j]], sem, add=True)
                     for j in range(_K)]
            for j in range(_K):
                descs[j].wait()
            return carry

        lax.fori_loop(0, cpt, chunk, 0)
        plsc.subcore_barrier()

        @pl.when(cid == 0)
        def _():
            pltpu.sync_copy(accd.at[nsl], q0.at[nsl])

        @pl.when(cid == 1)
        def _():
            pltpu.sync_copy(accd.at[nsl], q1.at[nsl])

    return pl.kernel(
        body,
        out_type=(jax.ShapeDtypeStruct((n_pad, 1), f32),
                  jax.ShapeDtypeStruct((n_pad, 1), f32)),
        mesh=mesh,
        scratch_types=(pltpu.VMEM_SHARED((n_pad, 1), f32),
                       pltpu.VMEM((_K, _LANES), jnp.int32),
                       pltpu.VMEM((_LANES, 1), f32),
                       pltpu.SemaphoreType.DMA),
        compiler_params=pltpu.CompilerParams(use_tc_tiling_on_sc=False),
        interpret=interpret)


def _prep_body(x_ref, w_ref, q0_ref, q1_ref, o_ref):
    d = lax.rsqrt(q0_ref[...] + q1_ref[...] + 1.0)
    o_ref[...] = jnp.dot(x_ref[...], w_ref[...],
                         preferred_element_type=jnp.float32) * d


def _mid_body(p0_ref, p1_ref, hn_ref, q0_ref, q1_ref, b_ref, w_ref, o_ref):
    d = lax.rsqrt(q0_ref[...] + q1_ref[...] + 1.0)
    t = (p0_ref[...] + p1_ref[...] + hn_ref[...]) * d + b_ref[...]
    t = jnp.maximum(t, 0.0)
    o_ref[...] = jnp.dot(t, w_ref[...], preferred_element_type=jnp.float32) * d


def _fin_body(p0_ref, p1_ref, hn_ref, q0_ref, q1_ref, b_ref, o_ref):
    d = lax.rsqrt(q0_ref[...] + q1_ref[...] + 1.0)
    o_ref[...] = (p0_ref[...] + p1_ref[...] + hn_ref[...]) * d + b_ref[...]


def _row_spec(blk, width):
    return pl.BlockSpec((blk, width), lambda i: (i, 0))


def _full_spec(shape):
    return pl.BlockSpec(shape, lambda i: (0, 0))


def _tc_call(body, n, blk, in_arrays, in_specs, interpret=False):
    return pl.pallas_call(
        body,
        grid=(n // blk,),
        in_specs=in_specs,
        out_specs=_row_spec(blk, _F),
        out_shape=jax.ShapeDtypeStruct((n, _F), jnp.float32),
        interpret=interpret)(*in_arrays)


def kernel(x, edge_index, batch_index, W1, b1, W2, b2, W3, b3):
    f32 = jnp.float32
    n, seq = x.shape
    e = edge_index.shape[1]
    emb = W1.shape[1]
    out_d = W3.shape[1]

    n_pad = _cdiv(n + 1, _NS * 8) * _NS * 8
    rpt_n = n_pad // _NS
    rpt_e = _cdiv(e, _NC * _NS * _K * _LANES) * _K
    rows2d = rpt_e * _NC * _NS
    pad = rows2d * _LANES - e

    src2 = jnp.concatenate(
        [edge_index[0], jnp.zeros((pad,), jnp.int32)]).reshape(rows2d, _LANES)
    dst2 = jnp.concatenate(
        [edge_index[1], jnp.full((pad,), n, jnp.int32)]).reshape(rows2d, _LANES)

    xp = jnp.pad(x, ((0, 0), (0, _F - seq)))
    W1p = jnp.pad(W1, ((0, _F - seq), (0, _F - emb)))
    W2p = jnp.pad(W2, ((0, _F - emb), (0, _F - emb)))
    W3p = jnp.pad(W3, ((0, _F - emb), (0, _F - out_d)))
    b1p = jnp.pad(b1, (0, _F - emb)).reshape(1, _F)
    b2p = jnp.pad(b2, (0, _F - emb)).reshape(1, _F)
    b3p = jnp.pad(b3, (0, _F - out_d)).reshape(1, _F)

    zz = jnp.zeros((n_pad, _F), f32)
    zd = jnp.zeros((n_pad, 1), f32)
    ones = jnp.ones((_LANES, 1), f32)

    deg_fn = _deg_make(n_pad, rpt_e, rpt_n)
    seg_fn = _seg_make(n_pad, rpt_e, rpt_n, _F)

    dq0, dq1 = deg_fn(dst2, zd, ones)
    q0, q1 = dq0[:n], dq1[:n]

    blk = 2000
    hn1 = _tc_call(_prep_body, n, blk, (xp, W1p, q0, q1),
                   [_row_spec(blk, _F), _full_spec((_F, _F)),
                    _row_spec(blk, 1), _row_spec(blk, 1)])

    s0, s1 = seg_fn(hn1, src2, dst2, zz)
    hn2 = _tc_call(_mid_body, n, blk, (s0[:n], s1[:n], hn1, q0, q1, b1p, W2p),
                   [_row_spec(blk, _F)] * 3 + [_row_spec(blk, 1)] * 2 +
                   [_full_spec((1, _F)), _full_spec((_F, _F))])

    s0, s1 = seg_fn(hn2, src2, dst2, zz)
    hn3 = _tc_call(_mid_body, n, blk, (s0[:n], s1[:n], hn2, q0, q1, b2p, W3p),
                   [_row_spec(blk, _F)] * 3 + [_row_spec(blk, 1)] * 2 +
                   [_full_spec((1, _F)), _full_spec((_F, _F))])

    s0, s1 = seg_fn(hn3, src2, dst2, zz)
    outp = _tc_call(_fin_body, n, blk, (s0[:n], s1[:n], hn3, q0, q1, b3p),
                    [_row_spec(blk, _F)] * 3 + [_row_spec(blk, 1)] * 2 +
                    [_full_spec((1, _F))])

    return outp[:, :out_d]


# speedup vs baseline: 60.6970x; 1.5085x over previous
import functools

import jax
import jax.numpy as jnp
from jax import lax
from jax.experimental import pallas as pl
from jax.experimental.pallas import tpu as pltpu
from jax.experimental.pallas import tpu_sc as plsc

_NC = 2
_NS = 16
_LANES = 128
_K = 8
_F = 16
_PK = 8


def _cdiv(a, b):
    return -(-a // b)


@functools.lru_cache(maxsize=None)
def _seg_make(n_pad, rpt_e, rpt_n, feat):
    f32 = jnp.float32
    mesh = plsc.VectorSubcoreMesh(core_axis_name="c", subcore_axis_name="s",
                                  num_cores=_NC, num_subcores=_NS)
    cpt = rpt_e // _K

    def body(hn, src2, dst2, zz, p0, p1, acc, src_v, dst_v, rows_v, sem):
        cid = lax.axis_index("c")
        sid = lax.axis_index("s")
        nsl = pl.ds(sid * rpt_n, rpt_n)
        pltpu.sync_copy(zz.at[nsl], acc.at[nsl])
        plsc.subcore_barrier()
        row0 = (cid * _NS + sid) * rpt_e

        def chunk(g, carry):
            base = row0 + g * _K
            pltpu.sync_copy(src2.at[pl.ds(base, _K)], src_v)
            pltpu.sync_copy(dst2.at[pl.ds(base, _K)], dst_v)
            descs = [pltpu.async_copy(hn.at[src_v.at[j]], rows_v.at[j], sem)
                     for j in range(_K)]
            for j in range(_K):
                descs[j].wait()
                pltpu.sync_copy(rows_v.at[j], acc.at[dst_v.at[j]], add=True)
            return carry

        lax.fori_loop(0, cpt, chunk, 0)
        plsc.subcore_barrier()

        @pl.when(cid == 0)
        def _():
            pltpu.sync_copy(acc.at[nsl], p0.at[nsl])

        @pl.when(cid == 1)
        def _():
            pltpu.sync_copy(acc.at[nsl], p1.at[nsl])

    return pl.kernel(
        body,
        out_type=(jax.ShapeDtypeStruct((n_pad, feat), f32),
                  jax.ShapeDtypeStruct((n_pad, feat), f32)),
        mesh=mesh,
        scratch_types=(pltpu.VMEM_SHARED((n_pad, feat), f32),
                       pltpu.VMEM((_K, _LANES), jnp.int32),
                       pltpu.VMEM((_K, _LANES), jnp.int32),
                       pltpu.VMEM((_K, _LANES, feat), f32),
                       pltpu.SemaphoreType.DMA),
        compiler_params=pltpu.CompilerParams(use_tc_tiling_on_sc=False))


@functools.lru_cache(maxsize=None)
def _deg_make(n_pad, rpt_e, rpt_n, feat):
    f32 = jnp.float32
    mesh = plsc.VectorSubcoreMesh(core_axis_name="c", subcore_axis_name="s",
                                  num_cores=_NC, num_subcores=_NS)
    cpt = rpt_e // _K

    def body(dst2, zz, ones, q0, q1, accd, dst_v, ones_v, sem):
        cid = lax.axis_index("c")
        sid = lax.axis_index("s")
        nsl = pl.ds(sid * rpt_n, rpt_n)
        pltpu.sync_copy(zz.at[nsl], accd.at[nsl])
        pltpu.sync_copy(ones, ones_v)
        plsc.subcore_barrier()
        row0 = (cid * _NS + sid) * rpt_e

        def chunk(g, carry):
            base = row0 + g * _K
            pltpu.sync_copy(dst2.at[pl.ds(base, _K)], dst_v)
            descs = [pltpu.async_copy(ones_v, accd.at[dst_v.at[j]], sem, add=True)
                     for j in range(_K)]
            for j in range(_K):
                descs[j].wait()
            return carry

        lax.fori_loop(0, cpt, chunk, 0)
        plsc.subcore_barrier()

        @pl.when(cid == 0)
        def _():
            pltpu.sync_copy(accd.at[nsl], q0.at[nsl])

        @pl.when(cid == 1)
        def _():
            pltpu.sync_copy(accd.at[nsl], q1.at[nsl])

    return pl.kernel(
        body,
        out_type=(jax.ShapeDtypeStruct((n_pad, feat), f32),
                  jax.ShapeDtypeStruct((n_pad, feat), f32)),
        mesh=mesh,
        scratch_types=(pltpu.VMEM_SHARED((n_pad, feat), f32),
                       pltpu.VMEM((_K, _LANES), jnp.int32),
                       pltpu.VMEM((_LANES, feat), f32),
                       pltpu.SemaphoreType.DMA),
        compiler_params=pltpu.CompilerParams(use_tc_tiling_on_sc=False))


def _prep_body(x_ref, w_ref, q0_ref, q1_ref, hn_ref, d_ref):
    d = lax.rsqrt(q0_ref[...] + q1_ref[...] + 1.0)
    d_ref[...] = d
    hn_ref[...] = jnp.dot(x_ref[...], w_ref[...],
                          preferred_element_type=jnp.float32) * d


def _mid_body(p0_ref, p1_ref, hn_ref, d_ref, b_ref, w_ref, o_ref):
    d = d_ref[...]
    t = (p0_ref[...] + p1_ref[...] + hn_ref[...]) * d + b_ref[...]
    t = jnp.maximum(t, 0.0)
    o_ref[...] = jnp.dot(t, w_ref[...], preferred_element_type=jnp.float32) * d


def _fin_body(p0_ref, p1_ref, hn_ref, d_ref, b_ref, o_ref):
    o_ref[...] = (p0_ref[...] + p1_ref[...] + hn_ref[...]) * d_ref[...] + b_ref[...]


def _row_spec(blk):
    return pl.BlockSpec((blk, _PK * _F), lambda i: (i, 0))


def _full_spec(shape):
    return pl.BlockSpec(shape, lambda i: (0, 0))


def _tc_call(body, rows_pk, in_arrays, in_specs, n_out):
    blk = rows_pk // 4
    oshape = jax.ShapeDtypeStruct((rows_pk, _PK * _F), jnp.float32)
    out_shape = [oshape] * n_out if n_out > 1 else oshape
    out_specs = [_row_spec(blk)] * n_out if n_out > 1 else _row_spec(blk)
    return pl.pallas_call(
        body,
        grid=(4,),
        in_specs=in_specs,
        out_specs=out_specs,
        out_shape=out_shape)(*in_arrays)


def kernel(x, edge_index, batch_index, W1, b1, W2, b2, W3, b3):
    f32 = jnp.float32
    n, seq = x.shape
    e = edge_index.shape[1]
    emb = W1.shape[1]
    out_d = W3.shape[1]

    n_pad = _cdiv(n + 1, 1024) * 1024
    rpt_n = n_pad // _NS
    rows_pk = n_pad // _PK
    rpt_e = _cdiv(e, _NC * _NS * _K * _LANES) * _K
    rows2d = rpt_e * _NC * _NS
    pad = rows2d * _LANES - e

    src2 = jnp.concatenate(
        [edge_index[0], jnp.zeros((pad,), jnp.int32)]).reshape(rows2d, _LANES)
    dst2 = jnp.concatenate(
        [edge_index[1], jnp.full((pad,), n, jnp.int32)]).reshape(rows2d, _LANES)

    eye8 = jnp.eye(_PK, dtype=f32)
    xp = jnp.pad(x, ((0, n_pad - n), (0, _F - seq))).reshape(rows_pk, _PK * _F)
    W1b = jnp.kron(eye8, jnp.pad(W1, ((0, _F - seq), (0, _F - emb))))
    W2b = jnp.kron(eye8, jnp.pad(W2, ((0, _F - emb), (0, _F - emb))))
    W3b = jnp.kron(eye8, jnp.pad(W3, ((0, _F - emb), (0, _F - out_d))))
    b1b = jnp.tile(jnp.pad(b1, (0, _F - emb)), _PK).reshape(1, _PK * _F)
    b2b = jnp.tile(jnp.pad(b2, (0, _F - emb)), _PK).reshape(1, _PK * _F)
    b3b = jnp.tile(jnp.pad(b3, (0, _F - out_d)), _PK).reshape(1, _PK * _F)

    zz = jnp.zeros((n_pad, _F), f32)
    ones = jnp.ones((_LANES, _F), f32)

    deg_fn = _deg_make(n_pad, rpt_e, rpt_n, _F)
    seg_fn = _seg_make(n_pad, rpt_e, rpt_n, _F)

    def pk(a):
        return a.reshape(rows_pk, _PK * _F)

    def unpk(a):
        return a.reshape(n_pad, _F)

    dq0, dq1 = deg_fn(dst2, zz, ones)

    hn1, dpk = _tc_call(_prep_body, rows_pk, (xp, W1b, pk(dq0), pk(dq1)),
                        [_row_spec(rows_pk // 4), _full_spec((_PK * _F, _PK * _F)),
                         _row_spec(rows_pk // 4), _row_spec(rows_pk // 4)], 2)

    s0, s1 = seg_fn(unpk(hn1), src2, dst2, zz)
    hn2 = _tc_call(_mid_body, rows_pk, (pk(s0), pk(s1), hn1, dpk, b1b, W2b),
                   [_row_spec(rows_pk // 4)] * 4 +
                   [_full_spec((1, _PK * _F)), _full_spec((_PK * _F, _PK * _F))], 1)

    s0, s1 = seg_fn(unpk(hn2), src2, dst2, zz)
    hn3 = _tc_call(_mid_body, rows_pk, (pk(s0), pk(s1), hn2, dpk, b2b, W3b),
                   [_row_spec(rows_pk // 4)] * 4 +
                   [_full_spec((1, _PK * _F)), _full_spec((_PK * _F, _PK * _F))], 1)

    s0, s1 = seg_fn(unpk(hn3), src2, dst2, zz)
    outp = _tc_call(_fin_body, rows_pk, (pk(s0), pk(s1), hn3, dpk, b3b),
                    [_row_spec(rows_pk // 4)] * 4 +
                    [_full_spec((1, _PK * _F))], 1)

    return unpk(outp)[:n, :out_d]
